# baseline (speedup 1.0000x reference)
import functools

import jax
import jax.numpy as jnp
from jax import lax
from jax.experimental import pallas as pl
from jax.experimental.pallas import tpu as pltpu
from jax.experimental.pallas import tpu_sc as plsc

NUM_ROWS = 1_000_000
DIM = 64
BATCH = 16384
HIST = 200
HALF = HIST // 2
PADH = 104
BUF = 2 * PADH
NVR = BUF // 16

NC = 2
NS = 16
NW = NC * NS
BPW = BATCH // NW
CHUNK = 256
NCHUNKS = BPW // CHUNK
NBUF = 16
NGROUPS = CHUNK // NBUF


def _tc_heads(table2, w4):
    half_rows = NUM_ROWS // 2
    rb = 8192
    grid = pl.cdiv(half_rows, rb)

    def body(t_ref, w_ref, o1e_ref, o1o_ref, o2e_ref, o2o_ref):
        r = lax.dot_general(w_ref[...], t_ref[...], (((1,), (1,)), ((), ())),
                            preferred_element_type=jnp.float32)
        o1e_ref[...] = r[0]
        o1o_ref[...] = r[1]
        o2e_ref[...] = r[2]
        o2o_ref[...] = r[3]

    out1d = jax.ShapeDtypeStruct((half_rows,), jnp.float32)
    return pl.pallas_call(
        body,
        grid=(grid,),
        in_specs=[
            pl.BlockSpec((rb, 2 * DIM), lambda i: (i, 0)),
            pl.BlockSpec((4, 2 * DIM), lambda i: (0, 0)),
        ],
        out_specs=[pl.BlockSpec((rb,), lambda i: (i,))] * 4,
        out_shape=[out1d] * 4,
    )(table2, w4)


def _sc_gather_reduce(x3, p1, p2, bias_vec):
    mesh = plsc.VectorSubcoreMesh(core_axis_name="c", subcore_axis_name="s",
                                  num_cores=NC, num_subcores=NS)

    @functools.partial(
        pl.kernel,
        out_type=[jax.ShapeDtypeStruct((BATCH,), jnp.float32),
                  jax.ShapeDtypeStruct((BATCH,), jnp.float32)],
        mesh=mesh,
        scratch_types=[
            pltpu.VMEM((CHUNK, 2, HALF), jnp.int32),
            pltpu.VMEM((NBUF, BUF), jnp.float32),
            pltpu.VMEM((NBUF, BUF), jnp.float32),
            pltpu.VMEM((CHUNK,), jnp.float32),
            pltpu.VMEM((CHUNK,), jnp.float32),
            pltpu.VMEM((16,), jnp.float32),
            pltpu.SemaphoreType.DMA((NBUF,)),
        ],
        compiler_params=pltpu.CompilerParams(use_tc_tiling_on_sc=False,
                                             needs_layout_passes=False),
    )
    def body(x_hbm, p1_hbm, p2_hbm, bias_hbm, out1_hbm, out2_hbm,
             idx_v, buf1_v, buf2_v, o1_v, o2_v, bias_v, sems):
        wid = lax.axis_index("s") * NC + lax.axis_index("c")
        base = wid * BPW
        pltpu.sync_copy(bias_hbm, bias_v)
        bv = bias_v[...]
        b1s = bv[0]
        b2s = bv[1]
        lanes = lax.iota(jnp.int32, 16)

        zeros16 = jnp.broadcast_to(jnp.float32(0.0), (16,))
        for s in range(NBUF):
            for j in range(NVR):
                buf1_v[s, pl.ds(16 * j, 16)] = zeros16
                buf2_v[s, pl.ds(16 * j, 16)] = zeros16

        def gathers(li, s):
            return [
                (p1_hbm.at[idx_v.at[li, 0]], buf1_v.at[s].at[pl.ds(0, HALF)]),
                (p1_hbm.at[idx_v.at[li, 1]], buf1_v.at[s].at[pl.ds(PADH, HALF)]),
                (p2_hbm.at[idx_v.at[li, 0]], buf2_v.at[s].at[pl.ds(0, HALF)]),
                (p2_hbm.at[idx_v.at[li, 1]], buf2_v.at[s].at[pl.ds(PADH, HALF)]),
            ]

        def issue(li, s):
            for src, dst in gathers(li, s):
                pltpu.async_copy(src, dst, sems.at[s])

        def drain(li, s):
            for src, dst in gathers(li, s):
                pltpu.make_async_copy(src, dst, sems.at[s]).wait()

        def reduce(s, v1, v2):
            acc1 = buf1_v[s, pl.ds(0, 16)]
            acc2 = buf2_v[s, pl.ds(0, 16)]
            for j in range(1, NVR):
                acc1 = acc1 + buf1_v[s, pl.ds(16 * j, 16)]
                acc2 = acc2 + buf2_v[s, pl.ds(16 * j, 16)]
            s1 = jnp.sum(acc1) + b1s
            s2 = jnp.sum(acc2) + b2s
            sel = lanes == s
            v1 = jnp.where(sel, jnp.broadcast_to(s1, (16,)), v1)
            v2 = jnp.where(sel, jnp.broadcast_to(s2, (16,)), v2)
            return v1, v2

        def chunk_body(ci, _):
            cbase = base + ci * CHUNK
            pltpu.sync_copy(x_hbm.at[pl.ds(cbase, CHUNK)], idx_v)
            for b in range(NBUF):
                issue(b, b)

            def group_body(g, _):
                v1 = zeros16
                v2 = zeros16
                for b in range(NBUF):
                    li = g * NBUF + b
                    drain(li, b)
                    v1, v2 = reduce(b, v1, v2)

                    @pl.when(li + NBUF < CHUNK)
                    def _():
                        issue(li + NBUF, b)
                o1_v[pl.ds(g * NBUF, 16)] = v1
                o2_v[pl.ds(g * NBUF, 16)] = v2
                return 0

            lax.fori_loop(0, NGROUPS, group_body, 0)
            pltpu.sync_copy(o1_v, out1_hbm.at[pl.ds(cbase, CHUNK)])
            pltpu.sync_copy(o2_v, out2_hbm.at[pl.ds(cbase, CHUNK)])
            return 0

        lax.fori_loop(0, NCHUNKS, chunk_body, 0)

    return body(x3, p1, p2, bias_vec)


@jax.jit
def kernel(x, table, W1, b1, W2, b2):
    s = 1.0 / HIST
    z = jnp.zeros((1, DIM), jnp.float32)
    w4 = jnp.concatenate([
        jnp.concatenate([W1 * s, z], axis=1),
        jnp.concatenate([z, W1 * s], axis=1),
        jnp.concatenate([W2 * s, z], axis=1),
        jnp.concatenate([z, W2 * s], axis=1),
    ], axis=0)
    bias_vec = jnp.concatenate(
        [b1, b2, jnp.zeros((14,), jnp.float32)])
    table2 = table.reshape(NUM_ROWS // 2, 2 * DIM)
    q1e, q1o, q2e, q2o = _tc_heads(table2, w4)
    p1 = jnp.concatenate([q1e, q1o])
    p2 = jnp.concatenate([q2e, q2o])
    x = x.astype(jnp.int32)
    pos = (x & 1) * (NUM_ROWS // 2) + (x >> 1)
    x3 = pos.reshape(BATCH, 2, HALF)
    out1, out2 = _sc_gather_reduce(x3, p1, p2, bias_vec)
    return (out1, out2)
